# baseline (speedup 1.0000x reference)
import functools

import jax
import jax.numpy as jnp
from jax.experimental import pallas as pl

N_NODES = 10000
IN_CH = 64
HEADS = 4
OUT_CH = 32
D_GAT = HEADS * OUT_CH
DIAG_LEN = 2000
MED_LEN = 500
OUTPUT_DIM = 500
DAY_DIM = 2 * D_GAT
RNN_H = 300
B = 32
T = 12
E = 320000


def _gat_conv(h_in, src, dst, W, a_s, a_d, b):
    N = h_in.shape[0]
    h = h_in @ W
    hh = h.reshape(N, HEADS, OUT_CH)
    a_src = jnp.sum(hh * a_s[None], axis=-1)
    a_dst = jnp.sum(hh * a_d[None], axis=-1)
    e = a_src[src] + a_dst[dst]
    e = jnp.where(e >= 0, e, 0.2 * e)
    p = jnp.exp(e)
    den = jax.ops.segment_sum(p, dst, num_segments=N)
    num = jax.ops.segment_sum(hh[src] * p[:, :, None], dst, num_segments=N)
    out = num / (den[:, :, None] + 1e-16)
    return out.reshape(N, D_GAT) + b


def _gat(x0, ei, i, W1, as1, ad1, b1, W2, as2, ad2, b2):
    loop = jnp.arange(N_NODES, dtype=ei.dtype)
    src = jnp.concatenate([ei[0], loop])
    dst = jnp.concatenate([ei[1], loop])
    g = jax.nn.relu(_gat_conv(x0, src, dst, W1[i], as1[i], ad1[i], b1[i]))
    return jax.nn.relu(_gat_conv(g, src, dst, W2[i], as2[i], ad2[i], b2[i]))


def _dipole(xs, Wih_f, Whh_f, bih_f, bhh_f, Wih_r, Whh_r, bih_r, bhh_r,
            W_attn, b_attn, W_ao, b_ao, W_out, b_out):
    Tn, Bn = xs.shape[0], xs.shape[1]

    def gru_step(h, inp, Wih, Whh, bih, bhh):
        xt, mask = inp
        gi = xt @ Wih.T + bih
        gh = h @ Whh.T + bhh
        ir, iz, inn = jnp.split(gi, 3, axis=-1)
        hr, hz, hn = jnp.split(gh, 3, axis=-1)
        r = jax.nn.sigmoid(ir + hr)
        z = jax.nn.sigmoid(iz + hz)
        n = jnp.tanh(inn + r * hn)
        hnew = (1.0 - z) * n + z * h
        hnew = jnp.where(mask, hnew, h)
        return hnew, hnew

    h0 = jnp.zeros((Bn, RNN_H), xs.dtype)
    ones = jnp.ones((Tn, Bn, 1), bool)
    _, hf = jax.lax.scan(
        functools.partial(gru_step, Wih=Wih_f, Whh=Whh_f, bih=bih_f, bhh=bhh_f),
        h0, (xs, ones))

    t_idx = jnp.arange(Tn)
    s_idx = jnp.arange(Tn)
    gather_t = jnp.clip(t_idx[None, :] - s_idx[:, None], 0, Tn - 1)
    xb = xs[gather_t]
    mask_b = (s_idx[:, None] <= t_idx[None, :])[:, :, None, None]
    h0b = jnp.zeros((Tn, Bn, RNN_H), xs.dtype)
    _, hb_all = jax.lax.scan(
        functools.partial(gru_step, Wih=Wih_r, Whh=Whh_r, bih=bih_r, bhh=bhh_r),
        h0b, (xb.reshape(Tn, Tn, Bn, -1), mask_b))

    j_idx = jnp.arange(Tn)
    s_for = jnp.clip(t_idx[:, None] - j_idx[None, :], 0, Tn - 1)
    hb_tj = hb_all[s_for, t_idx[:, None]]
    hf_tj = jnp.broadcast_to(hf[None], (Tn, Tn, Bn, RNN_H))
    rh = jnp.concatenate([hf_tj, hb_tj], axis=-1)
    valid = (j_idx[None, :] <= t_idx[:, None])[:, :, None]

    alpha = jnp.einsum("tjbh,h->tjb", rh, W_attn[0]) + b_attn[0]
    alpha = jnp.where(valid, alpha, -jnp.inf)
    alpha = jax.nn.softmax(alpha, axis=1)
    cnt = (t_idx + 1).astype(xs.dtype)[:, None, None]
    c = jnp.einsum("tjb,tjbh->tbh", alpha, jnp.where(valid[..., None], rh, 0.0)) / cnt
    rh_last = rh[t_idx, t_idx]
    ht = jnp.concatenate([c, rh_last], axis=-1)
    outs = ht @ W_ao.T + b_ao
    return jax.nn.sigmoid(outs @ W_out.T + b_out)


def kernel(x, ei_disease_drug_indication, ei_icd_tree, ei_atc_tree,
           ei_disease_drug_side, ei_drug_drug_edge, embedding_0,
           gat_W1, gat_asrc1, gat_adst1, gat_b1, gat_W2, gat_asrc2,
           gat_adst2, gat_b2, W_PM, b_PM, W_NM, b_NM, Wih_f, Whh_f, bih_f,
           bhh_f, Wih_r, Whh_r, bih_r, bhh_r, W_attn, b_attn, W_ao, b_ao,
           W_out, b_out):
    gp = (gat_W1, gat_asrc1, gat_adst1, gat_b1, gat_W2, gat_asrc2,
          gat_adst2, gat_b2)
    e_pos = jnp.stack([
        _gat(embedding_0, ei_disease_drug_indication, 0, *gp),
        _gat(embedding_0, ei_icd_tree, 1, *gp),
        _gat(embedding_0, ei_atc_tree, 2, *gp)], axis=1)
    e_neg = jnp.stack([
        _gat(embedding_0, ei_disease_drug_side, 3, *gp),
        _gat(embedding_0, ei_drug_drug_edge, 4, *gp)], axis=1)
    beta_p = jax.nn.relu((e_pos @ W_PM.T + b_PM)[..., 0])
    beta_p = jax.nn.softmax(beta_p, axis=1)[:, :, None]
    beta_n = jax.nn.relu((e_neg @ W_NM.T + b_NM)[..., 0])
    beta_n = jax.nn.softmax(beta_n, axis=1)[:, :, None]
    attn_applied = jnp.concatenate(
        [jnp.mean(beta_p * e_pos, axis=1), jnp.mean(beta_n * e_neg, axis=1)],
        axis=-1)
    xm = (x.reshape(-1, x.shape[2]) @ attn_applied[:DIAG_LEN + MED_LEN, :]
          ).reshape(x.shape[0], x.shape[1], -1)
    y = _dipole(jnp.transpose(xm, (1, 0, 2)), Wih_f, Whh_f, bih_f, bhh_f,
                Wih_r, Whh_r, bih_r, bhh_r, W_attn, b_attn, W_ao, b_ao,
                W_out, b_out)
    return (jnp.transpose(y, (1, 0, 2)), attn_applied)


# speedup vs baseline: 37.0723x; 37.0723x over previous
import functools

import jax
import jax.numpy as jnp
from jax import lax
from jax.experimental import pallas as pl
from jax.experimental.pallas import tpu as pltpu
from jax.experimental.pallas import tpu_sc as plsc

N_NODES = 10000
IN_CH = 64
HEADS = 4
OUT_CH = 32
D_GAT = HEADS * OUT_CH
DIAG_LEN = 2000
MED_LEN = 500
OUTPUT_DIM = 500
DAY_DIM = 2 * D_GAT
RNN_H = 300
B = 32
T = 12
E = 320000

NC, NS, L = 2, 16, 16
NP = 10112
ROWS_PER_TILE = NP // NS
HPC = HEADS // NC
DPC = HPC * OUT_CH
CHUNK = 64
EP_TOT = E + N_NODES
EDGES_PER_TILE = ((EP_TOT + NS - 1) // NS + CHUNK - 1) // CHUNK * CHUNK
CHUNKS_PER_TILE = EDGES_PER_TILE // CHUNK
E_PAD = EDGES_PER_TILE * NS


def _edge_kernel(h_hbm, asrc_hbm, adst_hbm, src_hbm, dst_hbm,
                 num_out, den_out,
                 asrc_v, adst_v, dentab_v, rows_v, src_v, dst_v, src2_v,
                 pgrp_v, num_acc, sem):
    core = lax.axis_index("c")
    sub = lax.axis_index("s")

    pltpu.sync_copy(asrc_hbm.at[core], asrc_v)
    pltpu.sync_copy(adst_hbm.at[core], adst_v)

    zero16 = jnp.zeros((L,), jnp.float32)

    def _z(i, carry):
        dentab_v[pl.ds(i * L, L)] = zero16
        return carry
    lax.fori_loop(0, NP * HPC // L, _z, 0)

    def _zr(c, carry):
        for k in range(DPC // L):
            rows_v[c, pl.ds(k * L, L)] = zero16
        return carry
    lax.fori_loop(0, CHUNK, _zr, 0)

    row0 = sub * ROWS_PER_TILE
    nfull = ROWS_PER_TILE // CHUNK
    tail = ROWS_PER_TILE - nfull * CHUNK
    for q in range(nfull):
        pltpu.sync_copy(rows_v, num_acc.at[pl.ds(row0 + q * CHUNK, CHUNK)])
    if tail:
        pltpu.sync_copy(rows_v.at[pl.ds(0, tail)],
                        num_acc.at[pl.ds(row0 + nfull * CHUNK, tail)])
    plsc.subcore_barrier()

    ebase = sub * EDGES_PER_TILE

    def _chunk(ci, carry):
        base = ebase + ci * CHUNK
        pltpu.sync_copy(src_hbm.at[pl.ds(base, CHUNK)], src_v)
        pltpu.sync_copy(dst_hbm.at[pl.ds(base, CHUNK)], dst_v)
        for g in range(CHUNK // L):
            src2_v[pl.ds(g * L, L)] = src_v[pl.ds(g * L, L)] * NC + core
        pltpu.async_copy(h_hbm.at[src2_v], rows_v, sem).wait()

        for g in range(CHUNK // L):
            s16 = src_v[pl.ds(g * L, L)]
            d16 = dst_v[pl.ds(g * L, L)]
            sb = s16 * HPC
            db = d16 * HPC
            for h in range(HPC):
                a_s = plsc.load_gather(asrc_v, [sb + h])
                a_d = plsc.load_gather(adst_v, [db + h])
                e = a_s + a_d
                e = jnp.where(e >= 0.0, e, 0.2 * e)
                p = jnp.exp(e)
                plsc.addupdate_scatter(dentab_v, [db + h], p)
                pgrp_v[h, pl.ds(g * L, L)] = p


        for c in range(CHUNK):
            for h in range(HPC):
                bc = plsc.load_gather(
                    pgrp_v, [jnp.full((L,), h, jnp.int32),
                             jnp.full((L,), c, jnp.int32)])
                for k in range(OUT_CH // L):
                    col = h * OUT_CH + k * L
                    rows_v[c, pl.ds(col, L)] = rows_v[c, pl.ds(col, L)] * bc

        pltpu.sync_copy(rows_v, num_acc.at[dst_v], add=True)
        return carry

    lax.fori_loop(0, CHUNKS_PER_TILE, _chunk, 0)
    pltpu.sync_copy(dentab_v, den_out.at[core, sub])
    plsc.subcore_barrier()

    for q in range(nfull):
        pltpu.sync_copy(num_acc.at[pl.ds(row0 + q * CHUNK, CHUNK)],
                        num_out.at[core, pl.ds(row0 + q * CHUNK, CHUNK)])
    if tail:
        pltpu.sync_copy(num_acc.at[pl.ds(row0 + nfull * CHUNK, tail)],
                        num_out.at[core, pl.ds(row0 + nfull * CHUNK, tail)])


_edge_call = functools.partial(
    pl.kernel,
    out_type=(jax.ShapeDtypeStruct((NC, NP, DPC), jnp.float32),
              jax.ShapeDtypeStruct((NC, NS, NP * HPC), jnp.float32)),
    mesh=plsc.VectorSubcoreMesh(core_axis_name="c", subcore_axis_name="s"),
    compiler_params=pltpu.CompilerParams(use_tc_tiling_on_sc=False, needs_layout_passes=False),
    scratch_types=[
        pltpu.VMEM((NP * HPC,), jnp.float32),
        pltpu.VMEM((NP * HPC,), jnp.float32),
        pltpu.VMEM((NP * HPC,), jnp.float32),
        pltpu.VMEM((CHUNK, DPC), jnp.float32),
        pltpu.VMEM((CHUNK,), jnp.int32),
        pltpu.VMEM((CHUNK,), jnp.int32),
        pltpu.VMEM((CHUNK,), jnp.int32),
        pltpu.VMEM((HPC, CHUNK), jnp.float32),
        pltpu.VMEM_SHARED((NP, DPC), jnp.float32),
        pltpu.SemaphoreType.DMA,
    ],
)(_edge_kernel)


def _gat_conv_sc(h_in, src_pad, dst_pad, W, a_s, a_d, b):
    h = h_in @ W
    hh = h.reshape(N_NODES, HEADS, OUT_CH)
    a_src = jnp.sum(hh * a_s[None], axis=-1)
    a_dst = jnp.sum(hh * a_d[None], axis=-1)
    hp = jnp.pad(h, ((0, NP - N_NODES), (0, 0)))
    hsplit = hp.reshape(NP * NC, DPC)
    asp = jnp.pad(a_src, ((0, NP - N_NODES), (0, 0)))
    asp = asp.reshape(NP, NC, HPC).transpose(1, 0, 2).reshape(NC, NP * HPC)
    adp = jnp.pad(a_dst, ((0, NP - N_NODES), (0, 0)))
    adp = adp.reshape(NP, NC, HPC).transpose(1, 0, 2).reshape(NC, NP * HPC)
    num2, den2 = _edge_call(hsplit, asp, adp, src_pad, dst_pad)
    num = jnp.concatenate([num2[0], num2[1]], axis=-1)[:N_NODES]
    den = den2.sum(1).reshape(NC, NP, HPC).transpose(1, 0, 2)
    den = den.reshape(NP, HEADS)[:N_NODES]
    out = num.reshape(N_NODES, HEADS, OUT_CH) / (den[:, :, None] + 1e-16)
    return out.reshape(N_NODES, D_GAT) + b


def _gat(x0, ei, i, W1, as1, ad1, b1, W2, as2, ad2, b2):
    loop = jnp.arange(N_NODES, dtype=ei.dtype)
    pad = jnp.full((E_PAD - EP_TOT,), N_NODES, ei.dtype)
    src = jnp.concatenate([ei[0], loop, pad])
    dst = jnp.concatenate([ei[1], loop, pad])
    g = jax.nn.relu(_gat_conv_sc(x0, src, dst, W1[i], as1[i], ad1[i], b1[i]))
    return jax.nn.relu(_gat_conv_sc(g, src, dst, W2[i], as2[i], ad2[i], b2[i]))


def _dipole(xs, Wih_f, Whh_f, bih_f, bhh_f, Wih_r, Whh_r, bih_r, bhh_r,
            W_attn, b_attn, W_ao, b_ao, W_out, b_out):
    Tn, Bn = xs.shape[0], xs.shape[1]

    def gru_step(h, inp, Wih, Whh, bih, bhh):
        xt, mask = inp
        gi = xt @ Wih.T + bih
        gh = h @ Whh.T + bhh
        ir, iz, inn = jnp.split(gi, 3, axis=-1)
        hr, hz, hn = jnp.split(gh, 3, axis=-1)
        r = jax.nn.sigmoid(ir + hr)
        z = jax.nn.sigmoid(iz + hz)
        n = jnp.tanh(inn + r * hn)
        hnew = (1.0 - z) * n + z * h
        hnew = jnp.where(mask, hnew, h)
        return hnew, hnew

    h0 = jnp.zeros((Bn, RNN_H), xs.dtype)
    ones = jnp.ones((Tn, Bn, 1), bool)
    _, hf = jax.lax.scan(
        functools.partial(gru_step, Wih=Wih_f, Whh=Whh_f, bih=bih_f, bhh=bhh_f),
        h0, (xs, ones))

    t_idx = jnp.arange(Tn)
    s_idx = jnp.arange(Tn)
    gather_t = jnp.clip(t_idx[None, :] - s_idx[:, None], 0, Tn - 1)
    xb = xs[gather_t]
    mask_b = (s_idx[:, None] <= t_idx[None, :])[:, :, None, None]
    h0b = jnp.zeros((Tn, Bn, RNN_H), xs.dtype)
    _, hb_all = jax.lax.scan(
        functools.partial(gru_step, Wih=Wih_r, Whh=Whh_r, bih=bih_r, bhh=bhh_r),
        h0b, (xb.reshape(Tn, Tn, Bn, -1), mask_b))

    j_idx = jnp.arange(Tn)
    s_for = jnp.clip(t_idx[:, None] - j_idx[None, :], 0, Tn - 1)
    hb_tj = hb_all[s_for, t_idx[:, None]]
    hf_tj = jnp.broadcast_to(hf[None], (Tn, Tn, Bn, RNN_H))
    rh = jnp.concatenate([hf_tj, hb_tj], axis=-1)
    valid = (j_idx[None, :] <= t_idx[:, None])[:, :, None]

    alpha = jnp.einsum("tjbh,h->tjb", rh, W_attn[0]) + b_attn[0]
    alpha = jnp.where(valid, alpha, -jnp.inf)
    alpha = jax.nn.softmax(alpha, axis=1)
    cnt = (t_idx + 1).astype(xs.dtype)[:, None, None]
    c = jnp.einsum("tjb,tjbh->tbh", alpha, jnp.where(valid[..., None], rh, 0.0)) / cnt
    rh_last = rh[t_idx, t_idx]
    ht = jnp.concatenate([c, rh_last], axis=-1)
    outs = ht @ W_ao.T + b_ao
    return jax.nn.sigmoid(outs @ W_out.T + b_out)


def kernel(x, ei_disease_drug_indication, ei_icd_tree, ei_atc_tree,
           ei_disease_drug_side, ei_drug_drug_edge, embedding_0,
           gat_W1, gat_asrc1, gat_adst1, gat_b1, gat_W2, gat_asrc2,
           gat_adst2, gat_b2, W_PM, b_PM, W_NM, b_NM, Wih_f, Whh_f, bih_f,
           bhh_f, Wih_r, Whh_r, bih_r, bhh_r, W_attn, b_attn, W_ao, b_ao,
           W_out, b_out):
    gp = (gat_W1, gat_asrc1, gat_adst1, gat_b1, gat_W2, gat_asrc2,
          gat_adst2, gat_b2)
    e_pos = jnp.stack([
        _gat(embedding_0, ei_disease_drug_indication, 0, *gp),
        _gat(embedding_0, ei_icd_tree, 1, *gp),
        _gat(embedding_0, ei_atc_tree, 2, *gp)], axis=1)
    e_neg = jnp.stack([
        _gat(embedding_0, ei_disease_drug_side, 3, *gp),
        _gat(embedding_0, ei_drug_drug_edge, 4, *gp)], axis=1)
    beta_p = jax.nn.relu((e_pos @ W_PM.T + b_PM)[..., 0])
    beta_p = jax.nn.softmax(beta_p, axis=1)[:, :, None]
    beta_n = jax.nn.relu((e_neg @ W_NM.T + b_NM)[..., 0])
    beta_n = jax.nn.softmax(beta_n, axis=1)[:, :, None]
    attn_applied = jnp.concatenate(
        [jnp.mean(beta_p * e_pos, axis=1), jnp.mean(beta_n * e_neg, axis=1)],
        axis=-1)
    xm = (x.reshape(-1, x.shape[2]) @ attn_applied[:DIAG_LEN + MED_LEN, :]
          ).reshape(x.shape[0], x.shape[1], -1)
    y = _dipole(jnp.transpose(xm, (1, 0, 2)), Wih_f, Whh_f, bih_f, bhh_f,
                Wih_r, Whh_r, bih_r, bhh_r, W_attn, b_attn, W_ao, b_ao,
                W_out, b_out)
    return (jnp.transpose(y, (1, 0, 2)), attn_applied)


# speedup vs baseline: 37.7278x; 1.0177x over previous
import functools

import jax
import jax.numpy as jnp
from jax import lax
from jax.experimental import pallas as pl
from jax.experimental.pallas import tpu as pltpu
from jax.experimental.pallas import tpu_sc as plsc

N_NODES = 10000
IN_CH = 64
HEADS = 4
OUT_CH = 32
D_GAT = HEADS * OUT_CH
DIAG_LEN = 2000
MED_LEN = 500
OUTPUT_DIM = 500
DAY_DIM = 2 * D_GAT
RNN_H = 300
B = 32
T = 12
E = 320000

NC, NS, L = 2, 16, 16
NP = 10112
ROWS_PER_TILE = NP // NS
HPC = HEADS // NC
DPC = HPC * OUT_CH
CHUNK = 128
EP_TOT = E + N_NODES
EDGES_PER_TILE = ((EP_TOT + NS - 1) // NS + CHUNK - 1) // CHUNK * CHUNK
CHUNKS_PER_TILE = EDGES_PER_TILE // CHUNK
E_PAD = EDGES_PER_TILE * NS


def _edge_kernel(h_hbm, asrc_hbm, adst_hbm, src_hbm, dst_hbm,
                 num_out, den_out,
                 asrc_v, adst_v, dentab_v, rows_v, src_v, dst_v, src2_v,
                 pgrp_v, num_acc, sem):
    core = lax.axis_index("c")
    sub = lax.axis_index("s")

    pltpu.sync_copy(asrc_hbm.at[core], asrc_v)
    pltpu.sync_copy(adst_hbm.at[core], adst_v)

    zero16 = jnp.zeros((L,), jnp.float32)

    def _z(i, carry):
        dentab_v[pl.ds(i * L, L)] = zero16
        return carry
    lax.fori_loop(0, NP * HPC // L, _z, 0)

    def _zr(c, carry):
        for k in range(DPC // L):
            rows_v[c, pl.ds(k * L, L)] = zero16
        return carry
    lax.fori_loop(0, CHUNK, _zr, 0)

    row0 = sub * ROWS_PER_TILE
    nfull = ROWS_PER_TILE // CHUNK
    tail = ROWS_PER_TILE - nfull * CHUNK
    for q in range(nfull):
        pltpu.sync_copy(rows_v, num_acc.at[pl.ds(row0 + q * CHUNK, CHUNK)])
    if tail:
        pltpu.sync_copy(rows_v.at[pl.ds(0, tail)],
                        num_acc.at[pl.ds(row0 + nfull * CHUNK, tail)])
    plsc.subcore_barrier()

    ebase = sub * EDGES_PER_TILE

    def _chunk(ci, carry):
        base = ebase + ci * CHUNK
        pltpu.sync_copy(src_hbm.at[pl.ds(base, CHUNK)], src_v)
        pltpu.sync_copy(dst_hbm.at[pl.ds(base, CHUNK)], dst_v)
        for g in range(CHUNK // L):
            src2_v[pl.ds(g * L, L)] = src_v[pl.ds(g * L, L)] * NC + core
        pltpu.async_copy(h_hbm.at[src2_v], rows_v, sem).wait()

        for g in range(CHUNK // L):
            s16 = src_v[pl.ds(g * L, L)]
            d16 = dst_v[pl.ds(g * L, L)]
            sb = s16 * HPC
            db = d16 * HPC
            for h in range(HPC):
                a_s = plsc.load_gather(asrc_v, [sb + h])
                a_d = plsc.load_gather(adst_v, [db + h])
                e = a_s + a_d
                e = jnp.where(e >= 0.0, e, 0.2 * e)
                p = jnp.exp(e)
                plsc.addupdate_scatter(dentab_v, [db + h], p)
                pgrp_v[h, pl.ds(g * L, L)] = p


        for c in range(CHUNK):
            for h in range(HPC):
                bc = plsc.load_gather(
                    pgrp_v, [jnp.full((L,), h, jnp.int32),
                             jnp.full((L,), c, jnp.int32)])
                for k in range(OUT_CH // L):
                    col = h * OUT_CH + k * L
                    rows_v[c, pl.ds(col, L)] = rows_v[c, pl.ds(col, L)] * bc

        pltpu.sync_copy(rows_v, num_acc.at[dst_v], add=True)
        return carry

    lax.fori_loop(0, CHUNKS_PER_TILE, _chunk, 0)
    pltpu.sync_copy(dentab_v, den_out.at[core, sub])
    plsc.subcore_barrier()

    for q in range(nfull):
        pltpu.sync_copy(num_acc.at[pl.ds(row0 + q * CHUNK, CHUNK)],
                        num_out.at[core, pl.ds(row0 + q * CHUNK, CHUNK)])
    if tail:
        pltpu.sync_copy(num_acc.at[pl.ds(row0 + nfull * CHUNK, tail)],
                        num_out.at[core, pl.ds(row0 + nfull * CHUNK, tail)])


_edge_call = functools.partial(
    pl.kernel,
    out_type=(jax.ShapeDtypeStruct((NC, NP, DPC), jnp.float32),
              jax.ShapeDtypeStruct((NC, NS, NP * HPC), jnp.float32)),
    mesh=plsc.VectorSubcoreMesh(core_axis_name="c", subcore_axis_name="s"),
    compiler_params=pltpu.CompilerParams(use_tc_tiling_on_sc=False, needs_layout_passes=False),
    scratch_types=[
        pltpu.VMEM((NP * HPC,), jnp.float32),
        pltpu.VMEM((NP * HPC,), jnp.float32),
        pltpu.VMEM((NP * HPC,), jnp.float32),
        pltpu.VMEM((CHUNK, DPC), jnp.float32),
        pltpu.VMEM((CHUNK,), jnp.int32),
        pltpu.VMEM((CHUNK,), jnp.int32),
        pltpu.VMEM((CHUNK,), jnp.int32),
        pltpu.VMEM((HPC, CHUNK), jnp.float32),
        pltpu.VMEM_SHARED((NP, DPC), jnp.float32),
        pltpu.SemaphoreType.DMA,
    ],
)(_edge_kernel)


def _gat_conv_sc(h_in, src_pad, dst_pad, W, a_s, a_d, b):
    h = h_in @ W
    hh = h.reshape(N_NODES, HEADS, OUT_CH)
    a_src = jnp.sum(hh * a_s[None], axis=-1)
    a_dst = jnp.sum(hh * a_d[None], axis=-1)
    hp = jnp.pad(h, ((0, NP - N_NODES), (0, 0)))
    hsplit = hp.reshape(NP * NC, DPC)
    asp = jnp.pad(a_src, ((0, NP - N_NODES), (0, 0)))
    asp = asp.reshape(NP, NC, HPC).transpose(1, 0, 2).reshape(NC, NP * HPC)
    adp = jnp.pad(a_dst, ((0, NP - N_NODES), (0, 0)))
    adp = adp.reshape(NP, NC, HPC).transpose(1, 0, 2).reshape(NC, NP * HPC)
    num2, den2 = _edge_call(hsplit, asp, adp, src_pad, dst_pad)
    num = jnp.concatenate([num2[0], num2[1]], axis=-1)[:N_NODES]
    den = den2.sum(1).reshape(NC, NP, HPC).transpose(1, 0, 2)
    den = den.reshape(NP, HEADS)[:N_NODES]
    out = num.reshape(N_NODES, HEADS, OUT_CH) / (den[:, :, None] + 1e-16)
    return out.reshape(N_NODES, D_GAT) + b


def _gat(x0, ei, i, W1, as1, ad1, b1, W2, as2, ad2, b2):
    loop = jnp.arange(N_NODES, dtype=ei.dtype)
    pad = jnp.full((E_PAD - EP_TOT,), N_NODES, ei.dtype)
    src = jnp.concatenate([ei[0], loop, pad])
    dst = jnp.concatenate([ei[1], loop, pad])
    g = jax.nn.relu(_gat_conv_sc(x0, src, dst, W1[i], as1[i], ad1[i], b1[i]))
    return jax.nn.relu(_gat_conv_sc(g, src, dst, W2[i], as2[i], ad2[i], b2[i]))


def _dipole(xs, Wih_f, Whh_f, bih_f, bhh_f, Wih_r, Whh_r, bih_r, bhh_r,
            W_attn, b_attn, W_ao, b_ao, W_out, b_out):
    Tn, Bn = xs.shape[0], xs.shape[1]

    def gru_step(h, inp, Wih, Whh, bih, bhh):
        xt, mask = inp
        gi = xt @ Wih.T + bih
        gh = h @ Whh.T + bhh
        ir, iz, inn = jnp.split(gi, 3, axis=-1)
        hr, hz, hn = jnp.split(gh, 3, axis=-1)
        r = jax.nn.sigmoid(ir + hr)
        z = jax.nn.sigmoid(iz + hz)
        n = jnp.tanh(inn + r * hn)
        hnew = (1.0 - z) * n + z * h
        hnew = jnp.where(mask, hnew, h)
        return hnew, hnew

    h0 = jnp.zeros((Bn, RNN_H), xs.dtype)
    ones = jnp.ones((Tn, Bn, 1), bool)
    _, hf = jax.lax.scan(
        functools.partial(gru_step, Wih=Wih_f, Whh=Whh_f, bih=bih_f, bhh=bhh_f),
        h0, (xs, ones))

    t_idx = jnp.arange(Tn)
    s_idx = jnp.arange(Tn)
    gather_t = jnp.clip(t_idx[None, :] - s_idx[:, None], 0, Tn - 1)
    xb = xs[gather_t]
    mask_b = (s_idx[:, None] <= t_idx[None, :])[:, :, None, None]
    h0b = jnp.zeros((Tn, Bn, RNN_H), xs.dtype)
    _, hb_all = jax.lax.scan(
        functools.partial(gru_step, Wih=Wih_r, Whh=Whh_r, bih=bih_r, bhh=bhh_r),
        h0b, (xb.reshape(Tn, Tn, Bn, -1), mask_b))

    j_idx = jnp.arange(Tn)
    s_for = jnp.clip(t_idx[:, None] - j_idx[None, :], 0, Tn - 1)
    hb_tj = hb_all[s_for, t_idx[:, None]]
    hf_tj = jnp.broadcast_to(hf[None], (Tn, Tn, Bn, RNN_H))
    rh = jnp.concatenate([hf_tj, hb_tj], axis=-1)
    valid = (j_idx[None, :] <= t_idx[:, None])[:, :, None]

    alpha = jnp.einsum("tjbh,h->tjb", rh, W_attn[0]) + b_attn[0]
    alpha = jnp.where(valid, alpha, -jnp.inf)
    alpha = jax.nn.softmax(alpha, axis=1)
    cnt = (t_idx + 1).astype(xs.dtype)[:, None, None]
    c = jnp.einsum("tjb,tjbh->tbh", alpha, jnp.where(valid[..., None], rh, 0.0)) / cnt
    rh_last = rh[t_idx, t_idx]
    ht = jnp.concatenate([c, rh_last], axis=-1)
    outs = ht @ W_ao.T + b_ao
    return jax.nn.sigmoid(outs @ W_out.T + b_out)


def kernel(x, ei_disease_drug_indication, ei_icd_tree, ei_atc_tree,
           ei_disease_drug_side, ei_drug_drug_edge, embedding_0,
           gat_W1, gat_asrc1, gat_adst1, gat_b1, gat_W2, gat_asrc2,
           gat_adst2, gat_b2, W_PM, b_PM, W_NM, b_NM, Wih_f, Whh_f, bih_f,
           bhh_f, Wih_r, Whh_r, bih_r, bhh_r, W_attn, b_attn, W_ao, b_ao,
           W_out, b_out):
    gp = (gat_W1, gat_asrc1, gat_adst1, gat_b1, gat_W2, gat_asrc2,
          gat_adst2, gat_b2)
    e_pos = jnp.stack([
        _gat(embedding_0, ei_disease_drug_indication, 0, *gp),
        _gat(embedding_0, ei_icd_tree, 1, *gp),
        _gat(embedding_0, ei_atc_tree, 2, *gp)], axis=1)
    e_neg = jnp.stack([
        _gat(embedding_0, ei_disease_drug_side, 3, *gp),
        _gat(embedding_0, ei_drug_drug_edge, 4, *gp)], axis=1)
    beta_p = jax.nn.relu((e_pos @ W_PM.T + b_PM)[..., 0])
    beta_p = jax.nn.softmax(beta_p, axis=1)[:, :, None]
    beta_n = jax.nn.relu((e_neg @ W_NM.T + b_NM)[..., 0])
    beta_n = jax.nn.softmax(beta_n, axis=1)[:, :, None]
    attn_applied = jnp.concatenate(
        [jnp.mean(beta_p * e_pos, axis=1), jnp.mean(beta_n * e_neg, axis=1)],
        axis=-1)
    xm = (x.reshape(-1, x.shape[2]) @ attn_applied[:DIAG_LEN + MED_LEN, :]
          ).reshape(x.shape[0], x.shape[1], -1)
    y = _dipole(jnp.transpose(xm, (1, 0, 2)), Wih_f, Whh_f, bih_f, bhh_f,
                Wih_r, Whh_r, bih_r, bhh_r, W_attn, b_attn, W_ao, b_ao,
                W_out, b_out)
    return (jnp.transpose(y, (1, 0, 2)), attn_applied)
